# baseline (speedup 1.0000x reference)
import functools

import jax
import jax.numpy as jnp
from jax.experimental import pallas as pl
from jax.experimental.pallas import tpu as pltpu

_EPS = 1e-12


def _prep_body(g1c_ref, b1_ref, wd_ref, bd_ref, g2c_ref, b2c_ref,
               mem_ref, wk_ref, bk_ref, wv_ref, bv_ref, g3c_ref, b3_ref,
               wu_ref, bu_ref,
               wd_aug_ref, wu_aug_ref, keyg_t_ref, valc_ref, const_ref):
    H, D = wd_ref.shape

    wdg = wd_ref[...] * g1c_ref[...]
    wdc = wdg - jnp.mean(wdg, axis=1, keepdims=True)
    csum_c = jnp.sum(wdc, axis=0, keepdims=True)
    wdc2 = wdc - csum_c * (1.0 / H)
    wd_aug_ref[...] = jnp.concatenate(
        [wdc2, jnp.ones((H, 1), jnp.float32)], axis=1)

    cb = jnp.dot(b1_ref[...], wd_ref[...],
                 preferred_element_type=jnp.float32) + bd_ref[...]
    cbc = cb - jnp.mean(cb, axis=1, keepdims=True)
    const_ref[...] = jnp.concatenate(
        [jnp.pad(cbc, ((0, 0), (0, 128 - D))),
         jnp.zeros((7, 128), jnp.float32)], axis=0)

    mem = mem_ref[...]
    key = jnp.dot(mem, wk_ref[...],
                  preferred_element_type=jnp.float32) + bk_ref[...]
    keyg_t_ref[...] = key.T * g2c_ref[...]

    val = jnp.dot(mem, wv_ref[...],
                  preferred_element_type=jnp.float32) + bv_ref[...]
    slot_w = jnp.exp(jnp.dot(key, b2c_ref[...],
                             preferred_element_type=jnp.float32))
    valw = val * slot_w
    valc_ref[...] = valw - jnp.mean(valw, axis=1, keepdims=True)

    wug = wu_ref[...] * g3c_ref[...]
    bias_row = jnp.dot(b3_ref[...], wu_ref[...],
                       preferred_element_type=jnp.float32) + bu_ref[...]
    wu_aug_ref[...] = jnp.concatenate([wug, bias_row], axis=0)


def _ffn_body(x_ref, wd_aug_ref, wu_aug_ref, keyg_t_ref, valc_ref,
              const_ref, o_ref):
    H = x_ref.shape[1]
    D = wu_aug_ref.shape[0] - 1

    x = x_ref[...]

    raw = jnp.dot(x, wd_aug_ref[...], preferred_element_type=jnp.float32)
    m = raw[:, D:D + 1] * (1.0 / H)
    sqsum = jnp.sum(x * x, axis=-1, keepdims=True)
    v = sqsum * (1.0 / H) - m * m
    s = jax.lax.rsqrt(v + _EPS)

    cbc = const_ref[0:1, :D]
    dc = s * raw[:, :D] + cbc
    v2 = jnp.mean(dc * dc, axis=-1, keepdims=True)
    qs = dc * jax.lax.rsqrt(v2 + _EPS)

    logits = jnp.dot(qs, keyg_t_ref[...], preferred_element_type=jnp.float32)
    e = jnp.exp(logits)
    mc = jnp.dot(e, valc_ref[...], preferred_element_type=jnp.float32)

    v3 = jnp.mean(mc * mc, axis=-1, keepdims=True)
    z = mc * jax.lax.rsqrt(v3 + _EPS)
    z_aug = jnp.concatenate(
        [z, jnp.ones((z.shape[0], 1), jnp.float32)], axis=1)

    o_ref[...] = jnp.dot(z_aug, wu_aug_ref[...], preferred_element_type=jnp.float32)


@functools.partial(jax.jit, static_argnames=("block_rows", "interpret"))
def _run(x2d, g1, b1, W_down, b_down, g2, b2, memory, W_k, b_k, W_v, b_v,
         g3, b3, W_up, b_up, block_rows=1024, interpret=False):
    n, H = x2d.shape
    D = W_down.shape[1]
    M = memory.shape[0]

    def full(a):
        return pl.BlockSpec(a.shape, lambda *_: (0,) * a.ndim)

    prep_ins = (g1.reshape(-1, 1), b1.reshape(1, -1), W_down,
                b_down.reshape(1, -1), g2.reshape(-1, 1), b2.reshape(-1, 1),
                memory, W_k, b_k.reshape(1, -1), W_v, b_v.reshape(1, -1),
                g3.reshape(-1, 1), b3.reshape(1, -1), W_up, b_up.reshape(1, -1))

    wd_aug, wu_aug, keyg_t, valc, consts = pl.pallas_call(
        _prep_body,
        out_shape=[jax.ShapeDtypeStruct((H, D + 1), jnp.float32),
                   jax.ShapeDtypeStruct((D + 1, H), jnp.float32),
                   jax.ShapeDtypeStruct((D, M), jnp.float32),
                   jax.ShapeDtypeStruct((M, D), jnp.float32),
                   jax.ShapeDtypeStruct((8, 128), jnp.float32)],
        name="bert_ffn_prep",
        interpret=interpret,
    )(*prep_ins)

    grid = (n // block_rows,)
    main_ins = (wd_aug, wu_aug, keyg_t, valc, consts)

    return pl.pallas_call(
        _ffn_body,
        out_shape=jax.ShapeDtypeStruct((n, H), jnp.float32),
        grid=grid,
        in_specs=[pl.BlockSpec((block_rows, H), lambda i: (i, 0))]
                 + [full(a) for a in main_ins],
        out_specs=pl.BlockSpec((block_rows, H), lambda i: (i, 0)),
        compiler_params=pltpu.CompilerParams(
            dimension_semantics=("arbitrary",),
            vmem_limit_bytes=50 * 1024 * 1024,
        ),
        name="bert_ffn_memory",
        interpret=interpret,
    )(x2d, *main_ins)


def kernel(hidden_states, g1, b1, W_down, b_down, g2, b2, memory, W_k, b_k,
           W_v, b_v, g3, b3, W_up, b_up, layer_id):
    B, S, H = hidden_states.shape
    x2d = hidden_states.reshape(B * S, H)
    out = _run(x2d, g1, b1, W_down, b_down, g2, b2, memory, W_k, b_k,
               W_v, b_v, g3, b3, W_up, b_up)
    return out.reshape(B, S, H)


# speedup vs baseline: 1.1011x; 1.1011x over previous
import functools

import jax
import jax.numpy as jnp
from jax.experimental import pallas as pl
from jax.experimental.pallas import tpu as pltpu

_EPS = 1e-12


def _prep_body(g1c_ref, b1_ref, wd_ref, bd_ref, g2c_ref, b2c_ref,
               mem_ref, wk_ref, bk_ref, wv_ref, bv_ref, g3c_ref, b3_ref,
               wu_ref, bu_ref,
               wd_aug_ref, wu_aug_ref, keyg_t_ref, valc_ref, const_ref):
    H, D = wd_ref.shape

    wdg = wd_ref[...] * g1c_ref[...]
    wdc = wdg - jnp.mean(wdg, axis=1, keepdims=True)
    csum_c = jnp.sum(wdc, axis=0, keepdims=True)
    wdc2 = wdc - csum_c * (1.0 / H)
    wd_aug_ref[...] = jnp.concatenate(
        [wdc2, jnp.ones((H, 1), jnp.float32)], axis=1)

    cb = jnp.dot(b1_ref[...], wd_ref[...],
                 preferred_element_type=jnp.float32) + bd_ref[...]
    cbc = cb - jnp.mean(cb, axis=1, keepdims=True)
    const_ref[...] = jnp.concatenate(
        [jnp.pad(cbc, ((0, 0), (0, 128 - D))),
         jnp.zeros((7, 128), jnp.float32)], axis=0)

    mem = mem_ref[...]
    key = jnp.dot(mem, wk_ref[...],
                  preferred_element_type=jnp.float32) + bk_ref[...]
    keyg_t_ref[...] = key.T * g2c_ref[...]

    val = jnp.dot(mem, wv_ref[...],
                  preferred_element_type=jnp.float32) + bv_ref[...]
    slot_w = jnp.exp(jnp.dot(key, b2c_ref[...],
                             preferred_element_type=jnp.float32))
    valw = val * slot_w
    valc_ref[...] = valw - jnp.mean(valw, axis=1, keepdims=True)

    wug = wu_ref[...] * g3c_ref[...]
    bias_row = jnp.dot(b3_ref[...], wu_ref[...],
                       preferred_element_type=jnp.float32) + bu_ref[...]
    wu_aug_ref[...] = jnp.concatenate([wug, bias_row], axis=0)


_N_CHUNKS = 4


def _ffn_body(x_ref, wd_aug_ref, wu_aug_ref, keyg_t_ref, valc_ref,
              const_ref, o_ref):
    H = x_ref.shape[1]
    D = wu_aug_ref.shape[0] - 1
    R = x_ref.shape[0]
    cs = R // _N_CHUNKS

    wd_aug = wd_aug_ref[...]
    wu_aug = wu_aug_ref[...]
    keyg_t = keyg_t_ref[...]
    valc = valc_ref[...]
    cbc = const_ref[0:1, :D]

    sl = [slice(c * cs, (c + 1) * cs) for c in range(_N_CHUNKS)]
    xs = [x_ref[r, :] for r in sl]
    raws = [jnp.dot(x, wd_aug, preferred_element_type=jnp.float32) for x in xs]
    sqs = [jnp.sum(x * x, axis=-1, keepdims=True) for x in xs]
    ms = [r[:, D:D + 1] * (1.0 / H) for r in raws]
    ss = [jax.lax.rsqrt(sq * (1.0 / H) - m * m + _EPS)
          for sq, m in zip(sqs, ms)]
    dcs = [s * r[:, :D] + cbc for s, r in zip(ss, raws)]
    v2s = [jnp.mean(dc * dc, axis=-1, keepdims=True) for dc in dcs]
    qss = [dc * jax.lax.rsqrt(v2 + _EPS) for dc, v2 in zip(dcs, v2s)]
    ls = [jnp.dot(q, keyg_t, preferred_element_type=jnp.float32) for q in qss]
    es = [jnp.exp(l) for l in ls]
    mcs = [jnp.dot(e, valc, preferred_element_type=jnp.float32) for e in es]
    v3s = [jnp.mean(mc * mc, axis=-1, keepdims=True) for mc in mcs]
    zs = [mc * jax.lax.rsqrt(v3 + _EPS) for mc, v3 in zip(mcs, v3s)]
    zas = [jnp.concatenate([z, jnp.ones((cs, 1), jnp.float32)], axis=1)
           for z in zs]
    for r, za in zip(sl, zas):
        o_ref[r, :] = jnp.dot(za, wu_aug, preferred_element_type=jnp.float32)


@functools.partial(jax.jit, static_argnames=("block_rows", "interpret"))
def _run(x2d, g1, b1, W_down, b_down, g2, b2, memory, W_k, b_k, W_v, b_v,
         g3, b3, W_up, b_up, block_rows=1024, interpret=False):
    n, H = x2d.shape
    D = W_down.shape[1]
    M = memory.shape[0]

    def full(a):
        return pl.BlockSpec(a.shape, lambda *_: (0,) * a.ndim)

    prep_ins = (g1.reshape(-1, 1), b1.reshape(1, -1), W_down,
                b_down.reshape(1, -1), g2.reshape(-1, 1), b2.reshape(-1, 1),
                memory, W_k, b_k.reshape(1, -1), W_v, b_v.reshape(1, -1),
                g3.reshape(-1, 1), b3.reshape(1, -1), W_up, b_up.reshape(1, -1))

    wd_aug, wu_aug, keyg_t, valc, consts = pl.pallas_call(
        _prep_body,
        out_shape=[jax.ShapeDtypeStruct((H, D + 1), jnp.float32),
                   jax.ShapeDtypeStruct((D + 1, H), jnp.float32),
                   jax.ShapeDtypeStruct((D, M), jnp.float32),
                   jax.ShapeDtypeStruct((M, D), jnp.float32),
                   jax.ShapeDtypeStruct((8, 128), jnp.float32)],
        name="bert_ffn_prep",
        interpret=interpret,
    )(*prep_ins)

    grid = (n // block_rows,)
    main_ins = (wd_aug, wu_aug, keyg_t, valc, consts)

    return pl.pallas_call(
        _ffn_body,
        out_shape=jax.ShapeDtypeStruct((n, H), jnp.float32),
        grid=grid,
        in_specs=[pl.BlockSpec((block_rows, H), lambda i: (i, 0))]
                 + [full(a) for a in main_ins],
        out_specs=pl.BlockSpec((block_rows, H), lambda i: (i, 0)),
        compiler_params=pltpu.CompilerParams(
            dimension_semantics=("arbitrary",),
            vmem_limit_bytes=50 * 1024 * 1024,
        ),
        name="bert_ffn_memory",
        interpret=interpret,
    )(x2d, *main_ins)


def kernel(hidden_states, g1, b1, W_down, b_down, g2, b2, memory, W_k, b_k,
           W_v, b_v, g3, b3, W_up, b_up, layer_id):
    B, S, H = hidden_states.shape
    x2d = hidden_states.reshape(B * S, H)
    out = _run(x2d, g1, b1, W_down, b_down, g2, b2, memory, W_k, b_k,
               W_v, b_v, g3, b3, W_up, b_up)
    return out.reshape(B, S, H)


# speedup vs baseline: 1.1524x; 1.0467x over previous
import functools

import jax
import jax.numpy as jnp
from jax.experimental import pallas as pl
from jax.experimental.pallas import tpu as pltpu

_EPS = 1e-12


def _prep_body(g1_ref, b1_ref, wd_ref, bd_ref, g2_ref, b2_ref,
               mem_ref, wk_ref, bk_ref, wv_ref, bv_ref, g3_ref, b3_ref,
               wu_ref, bu_ref,
               wd_aug_ref, wu_aug_ref, keyg_t_ref, valc_ref, const_ref):
    H, D = wd_ref.shape

    wdg = wd_ref[...] * g1_ref[...].T
    wdc = wdg - jnp.mean(wdg, axis=1, keepdims=True)
    csum_c = jnp.sum(wdc, axis=0, keepdims=True)
    wdc2 = wdc - csum_c * (1.0 / H)
    wd_aug_ref[...] = jnp.concatenate(
        [wdc2, jnp.ones((H, 1), jnp.float32)], axis=1)

    cb = jnp.dot(b1_ref[...], wd_ref[...],
                 preferred_element_type=jnp.float32) + bd_ref[...]
    cbc = cb - jnp.mean(cb, axis=1, keepdims=True)
    const_ref[...] = jnp.concatenate(
        [jnp.pad(cbc, ((0, 0), (0, 128 - D))),
         jnp.zeros((7, 128), jnp.float32)], axis=0)

    mem = mem_ref[...]
    key = jnp.dot(mem, wk_ref[...],
                  preferred_element_type=jnp.float32) + bk_ref[...]
    keyg_t_ref[...] = (key * g2_ref[...]).T

    val = jnp.dot(mem, wv_ref[...],
                  preferred_element_type=jnp.float32) + bv_ref[...]
    slot_w = jnp.exp(jax.lax.dot_general(
        key, b2_ref[...], (((1,), (1,)), ((), ())),
        preferred_element_type=jnp.float32))
    valw = val * slot_w
    valc_ref[...] = valw - jnp.mean(valw, axis=1, keepdims=True)

    wug = wu_ref[...] * g3_ref[...].T
    bias_row = jnp.dot(b3_ref[...], wu_ref[...],
                       preferred_element_type=jnp.float32) + bu_ref[...]
    wu_aug_ref[...] = jnp.concatenate([wug, bias_row], axis=0)


_N_CHUNKS = 4


def _ffn_body(x_ref, wd_aug_ref, wu_aug_ref, keyg_t_ref, valc_ref,
              const_ref, o_ref):
    H = x_ref.shape[1]
    D = wu_aug_ref.shape[0] - 1
    R = x_ref.shape[0]
    cs = R // _N_CHUNKS

    wd_aug = wd_aug_ref[...]
    wu_aug = wu_aug_ref[...]
    keyg_t = keyg_t_ref[...]
    valc = valc_ref[...]
    cbc = const_ref[0:1, :D]

    sl = [slice(c * cs, (c + 1) * cs) for c in range(_N_CHUNKS)]
    xs = [x_ref[r, :] for r in sl]
    raws = [jnp.dot(x, wd_aug, preferred_element_type=jnp.float32) for x in xs]
    sqs = [jnp.sum(x * x, axis=-1, keepdims=True) for x in xs]
    ms = [r[:, D:D + 1] * (1.0 / H) for r in raws]
    ss = [jax.lax.rsqrt(sq * (1.0 / H) - m * m + _EPS)
          for sq, m in zip(sqs, ms)]
    dcs = [s * r[:, :D] + cbc for s, r in zip(ss, raws)]
    v2s = [jnp.mean(dc * dc, axis=-1, keepdims=True) for dc in dcs]
    qss = [dc * jax.lax.rsqrt(v2 + _EPS) for dc, v2 in zip(dcs, v2s)]
    ls = [jnp.dot(q, keyg_t, preferred_element_type=jnp.float32) for q in qss]
    es = [jnp.exp(l) for l in ls]
    mcs = [jnp.dot(e, valc, preferred_element_type=jnp.float32) for e in es]
    v3s = [jnp.mean(mc * mc, axis=-1, keepdims=True) for mc in mcs]
    zs = [mc * jax.lax.rsqrt(v3 + _EPS) for mc, v3 in zip(mcs, v3s)]
    zas = [jnp.concatenate([z, jnp.ones((cs, 1), jnp.float32)], axis=1)
           for z in zs]
    for r, za in zip(sl, zas):
        o_ref[r, :] = jnp.dot(za, wu_aug, preferred_element_type=jnp.float32)


@functools.partial(jax.jit, static_argnames=("block_rows", "interpret"))
def _run(x2d, g1, b1, W_down, b_down, g2, b2, memory, W_k, b_k, W_v, b_v,
         g3, b3, W_up, b_up, block_rows=1024, interpret=False):
    n, H = x2d.shape
    D = W_down.shape[1]
    M = memory.shape[0]

    def full(a):
        return pl.BlockSpec(a.shape, lambda *_: (0,) * a.ndim)

    prep_ins = (g1.reshape(1, -1), b1.reshape(1, -1), W_down,
                b_down.reshape(1, -1), g2.reshape(1, -1), b2.reshape(1, -1),
                memory, W_k, b_k.reshape(1, -1), W_v, b_v.reshape(1, -1),
                g3.reshape(1, -1), b3.reshape(1, -1), W_up, b_up.reshape(1, -1))

    wd_aug, wu_aug, keyg_t, valc, consts = pl.pallas_call(
        _prep_body,
        out_shape=[jax.ShapeDtypeStruct((H, D + 1), jnp.float32),
                   jax.ShapeDtypeStruct((D + 1, H), jnp.float32),
                   jax.ShapeDtypeStruct((D, M), jnp.float32),
                   jax.ShapeDtypeStruct((M, D), jnp.float32),
                   jax.ShapeDtypeStruct((8, 128), jnp.float32)],
        name="bert_ffn_prep",
        interpret=interpret,
    )(*prep_ins)

    grid = (n // block_rows,)
    main_ins = (wd_aug, wu_aug, keyg_t, valc, consts)

    return pl.pallas_call(
        _ffn_body,
        out_shape=jax.ShapeDtypeStruct((n, H), jnp.float32),
        grid=grid,
        in_specs=[pl.BlockSpec((block_rows, H), lambda i: (i, 0))]
                 + [full(a) for a in main_ins],
        out_specs=pl.BlockSpec((block_rows, H), lambda i: (i, 0)),
        compiler_params=pltpu.CompilerParams(
            dimension_semantics=("arbitrary",),
            vmem_limit_bytes=50 * 1024 * 1024,
        ),
        name="bert_ffn_memory",
        interpret=interpret,
    )(x2d, *main_ins)


def kernel(hidden_states, g1, b1, W_down, b_down, g2, b2, memory, W_k, b_k,
           W_v, b_v, g3, b3, W_up, b_up, layer_id):
    B, S, H = hidden_states.shape
    x2d = hidden_states.reshape(B * S, H)
    out = _run(x2d, g1, b1, W_down, b_down, g2, b2, memory, W_k, b_k,
               W_v, b_v, g3, b3, W_up, b_up)
    return out.reshape(B, S, H)


# speedup vs baseline: 1.4087x; 1.2224x over previous
import functools

import jax
import jax.numpy as jnp
from jax.experimental import pallas as pl
from jax.experimental.pallas import tpu as pltpu

_EPS = 1e-12


def _prep_body(g1_ref, b1_ref, wd_ref, bd_ref, g2_ref, b2_ref,
               mem_ref, wk_ref, bk_ref, wv_ref, bv_ref, g3_ref, b3_ref,
               wu_ref, bu_ref,
               wd_aug_ref, wu_aug_ref, keyg_t_ref, valc_ref, const_ref):
    H, D = wd_ref.shape

    def row(ref):
        return ref[...].reshape(1, -1)

    g1_row, b1_row, bd_row = row(g1_ref), row(b1_ref), row(bd_ref)
    g2_row, b2_row, bk_row = row(g2_ref), row(b2_ref), row(bk_ref)
    bv_row, g3_row, b3_row, bu_row = (row(bv_ref), row(g3_ref),
                                      row(b3_ref), row(bu_ref))

    wdg = wd_ref[...] * g1_row.T
    wdc = wdg - jnp.mean(wdg, axis=1, keepdims=True)
    csum_c = jnp.sum(wdc, axis=0, keepdims=True)
    wdc2 = wdc - csum_c * (1.0 / H)
    wd_aug_ref[...] = jnp.concatenate(
        [wdc2, jnp.ones((H, 1), jnp.float32)], axis=1)

    cb = jnp.dot(b1_row, wd_ref[...],
                 preferred_element_type=jnp.float32) + bd_row
    cbc = cb - jnp.mean(cb, axis=1, keepdims=True)
    const_ref[...] = jnp.concatenate(
        [jnp.pad(cbc, ((0, 0), (0, 128 - D))),
         jnp.zeros((7, 128), jnp.float32)], axis=0)

    mem = mem_ref[...]
    key = jnp.dot(mem, wk_ref[...],
                  preferred_element_type=jnp.float32) + bk_row
    keyg_t_ref[...] = (key * g2_row).T

    val = jnp.dot(mem, wv_ref[...],
                  preferred_element_type=jnp.float32) + bv_row
    slot_w = jnp.exp(jax.lax.dot_general(
        key, b2_row, (((1,), (1,)), ((), ())),
        preferred_element_type=jnp.float32))
    valw = val * slot_w
    valc_ref[...] = valw - jnp.mean(valw, axis=1, keepdims=True)

    wug = wu_ref[...] * g3_row.T
    bias_row = jnp.dot(b3_row, wu_ref[...],
                       preferred_element_type=jnp.float32) + bu_row
    wu_aug_ref[...] = jnp.concatenate([wug, bias_row], axis=0)


def _ffn_body(x_ref, wd_aug_ref, wu_aug_ref, keyg_t_ref, valc_ref,
              const_ref, o_ref):
    n_chunks, cs, H = x_ref.shape
    D = wu_aug_ref.shape[0] - 1

    wd_aug = wd_aug_ref[...]
    wu_aug = wu_aug_ref[...]
    keyg_t = keyg_t_ref[...]
    valc = valc_ref[...]
    cbc = const_ref[0:1, :D]

    xs = [x_ref[c] for c in range(n_chunks)]
    raws = [jnp.dot(x, wd_aug, preferred_element_type=jnp.float32) for x in xs]
    sqs = [jnp.sum(x * x, axis=-1, keepdims=True) for x in xs]
    ms = [r[:, D:D + 1] * (1.0 / H) for r in raws]
    ss = [jax.lax.rsqrt(sq * (1.0 / H) - m * m + _EPS)
          for sq, m in zip(sqs, ms)]
    dcs = [s * r[:, :D] + cbc for s, r in zip(ss, raws)]
    v2s = [jnp.mean(dc * dc, axis=-1, keepdims=True) for dc in dcs]
    qss = [dc * jax.lax.rsqrt(v2 + _EPS) for dc, v2 in zip(dcs, v2s)]
    ls = [jnp.dot(q, keyg_t, preferred_element_type=jnp.float32) for q in qss]
    es = [jnp.exp(l) for l in ls]
    mcs = [jnp.dot(e, valc, preferred_element_type=jnp.float32) for e in es]
    v3s = [jnp.mean(mc * mc, axis=-1, keepdims=True) for mc in mcs]
    zs = [mc * jax.lax.rsqrt(v3 + _EPS) for mc, v3 in zip(mcs, v3s)]
    zas = [jnp.concatenate([z, jnp.ones((cs, 1), jnp.float32)], axis=1)
           for z in zs]
    for c, za in enumerate(zas):
        o_ref[c] = jnp.dot(za, wu_aug, preferred_element_type=jnp.float32)


@functools.partial(jax.jit, static_argnames=("block_b", "interpret"))
def _run(x3d, g1, b1, W_down, b_down, g2, b2, memory, W_k, b_k, W_v, b_v,
         g3, b3, W_up, b_up, block_b=4, interpret=False):
    B, S, H = x3d.shape
    D = W_down.shape[1]
    M = memory.shape[0]

    def full(a):
        return pl.BlockSpec(a.shape, lambda *_: (0,) * a.ndim)

    prep_ins = (g1, b1, W_down, b_down, g2, b2, memory, W_k, b_k,
                W_v, b_v, g3, b3, W_up, b_up)

    wd_aug, wu_aug, keyg_t, valc, consts = pl.pallas_call(
        _prep_body,
        out_shape=[jax.ShapeDtypeStruct((H, D + 1), jnp.float32),
                   jax.ShapeDtypeStruct((D + 1, H), jnp.float32),
                   jax.ShapeDtypeStruct((D, M), jnp.float32),
                   jax.ShapeDtypeStruct((M, D), jnp.float32),
                   jax.ShapeDtypeStruct((8, 128), jnp.float32)],
        name="bert_ffn_prep",
        interpret=interpret,
    )(*prep_ins)

    grid = (B // block_b,)
    main_ins = (wd_aug, wu_aug, keyg_t, valc, consts)

    return pl.pallas_call(
        _ffn_body,
        out_shape=jax.ShapeDtypeStruct((B, S, H), jnp.float32),
        grid=grid,
        in_specs=[pl.BlockSpec((block_b, S, H), lambda i: (i, 0, 0))]
                 + [full(a) for a in main_ins],
        out_specs=pl.BlockSpec((block_b, S, H), lambda i: (i, 0, 0)),
        compiler_params=pltpu.CompilerParams(
            dimension_semantics=("arbitrary",),
            vmem_limit_bytes=50 * 1024 * 1024,
        ),
        name="bert_ffn_memory",
        interpret=interpret,
    )(x3d, *main_ins)


def kernel(hidden_states, g1, b1, W_down, b_down, g2, b2, memory, W_k, b_k,
           W_v, b_v, g3, b3, W_up, b_up, layer_id):
    return _run(hidden_states, g1, b1, W_down, b_down, g2, b2, memory,
                W_k, b_k, W_v, b_v, g3, b3, W_up, b_up)
